# baseline (speedup 1.0000x reference)
import functools

import jax
import jax.numpy as jnp
from jax import lax
from jax.experimental import pallas as pl
from jax.experimental.pallas import tpu as pltpu
from jax.experimental.pallas import tpu_sc as plsc

B = 1024
CTX = 8
VOCAB = 100000
WDIM = 32
DDIM = 32
IN_FEAT = CTX * WDIM + DDIM

VT = 2048
NV = (VOCAB + VT - 1) // VT
VPAD = NV * VT

_NC = 2
_NS = 16
_NW = _NC * _NS
_BPW = B // _NW
_WPW = _BPW * CTX
_WCH = _WPW // 128


def _gather_body(doc_ids, word_ids, doc_emb, word_emb, pe_out, we_out,
                 didx_v, drows_v, widx_v, wrows_v, dsem, wsem):
    wid = lax.axis_index("s") * _NC + lax.axis_index("c")
    base = wid * _BPW
    wbase = wid * _WPW
    pltpu.sync_copy(doc_ids.at[pl.ds(base, _BPW)], didx_v)
    for j in range(_WCH):
        pltpu.sync_copy(word_ids.at[pl.ds(wbase + j * 128, 128)], widx_v.at[j])
    dcp = pltpu.async_copy(doc_emb.at[didx_v], drows_v, dsem)
    wcps = [
        pltpu.async_copy(word_emb.at[widx_v.at[j]],
                         wrows_v.at[pl.ds(j * 128, 128)], wsem)
        for j in range(_WCH)
    ]
    dcp.wait()
    pltpu.sync_copy(drows_v, pe_out.at[pl.ds(base, _BPW)])
    for c in wcps:
        c.wait()
    pltpu.sync_copy(wrows_v, we_out.at[pl.ds(wbase, _WPW)])


@functools.lru_cache(maxsize=1)
def _gather_kernel():
    return pl.kernel(
        _gather_body,
        mesh=plsc.VectorSubcoreMesh(core_axis_name="c", subcore_axis_name="s"),
        compiler_params=pltpu.CompilerParams(use_tc_tiling_on_sc=False),
        out_type=[
            jax.ShapeDtypeStruct((B, DDIM), jnp.float32),
            jax.ShapeDtypeStruct((B * CTX, WDIM), jnp.float32),
        ],
        scratch_types=[
            pltpu.VMEM((_BPW,), jnp.int32),
            pltpu.VMEM((_BPW, DDIM), jnp.float32),
            pltpu.VMEM((_WCH, 128), jnp.int32),
            pltpu.VMEM((_WPW, WDIM), jnp.float32),
            pltpu.SemaphoreType.DMA,
            pltpu.SemaphoreType.DMA,
        ],
    )


def _logits_tile(concat_ref, w_ref, b_ref):
    l = lax.dot_general(concat_ref[...], w_ref[...],
                        (((1,), (1,)), ((), ())),
                        preferred_element_type=jnp.float32)
    return l + b_ref[0]


def _stats_body(concat_ref, w_ref, b_ref, m_ref, s_ref, *, vt, vocab):
    j = pl.program_id(0)
    l = _logits_tile(concat_ref, w_ref, b_ref)
    col = j * vt + lax.broadcasted_iota(jnp.int32, l.shape, 1)
    l = jnp.where(col < vocab, l, -1e30)
    tmax = jnp.max(l, axis=1, keepdims=True)

    @pl.when(j == 0)
    def _():
        m_ref[...] = tmax
        s_ref[...] = jnp.sum(jnp.exp(l - tmax), axis=1, keepdims=True)

    @pl.when(j > 0)
    def _():
        m_old = m_ref[...]
        m_new = jnp.maximum(m_old, tmax)
        s_ref[...] = (s_ref[...] * jnp.exp(m_old - m_new)
                      + jnp.sum(jnp.exp(l - m_new), axis=1, keepdims=True))
        m_ref[...] = m_new


def _out_body(concat_ref, w_ref, b_ref, m_ref, s_ref, o_ref):
    l = _logits_tile(concat_ref, w_ref, b_ref)
    o_ref[...] = jnp.exp(l - m_ref[...]) * (1.0 / s_ref[...])


_stats_call = pl.pallas_call(
    functools.partial(_stats_body, vt=VT, vocab=VOCAB),
    grid=(NV,),
    in_specs=[
        pl.BlockSpec((B, IN_FEAT), lambda j: (0, 0)),
        pl.BlockSpec((VT, IN_FEAT), lambda j: (j, 0)),
        pl.BlockSpec((1, 1, VT), lambda j: (j, 0, 0)),
    ],
    out_specs=[
        pl.BlockSpec((B, 1), lambda j: (0, 0)),
        pl.BlockSpec((B, 1), lambda j: (0, 0)),
    ],
    out_shape=[
        jax.ShapeDtypeStruct((B, 1), jnp.float32),
        jax.ShapeDtypeStruct((B, 1), jnp.float32),
    ],
)

_out_call = pl.pallas_call(
    _out_body,
    grid=(NV,),
    in_specs=[
        pl.BlockSpec((B, IN_FEAT), lambda j: (0, 0)),
        pl.BlockSpec((VT, IN_FEAT), lambda j: (j, 0)),
        pl.BlockSpec((1, 1, VT), lambda j: (j, 0, 0)),
        pl.BlockSpec((B, 1), lambda j: (0, 0)),
        pl.BlockSpec((B, 1), lambda j: (0, 0)),
    ],
    out_specs=pl.BlockSpec((B, VT), lambda j: (0, j)),
    out_shape=jax.ShapeDtypeStruct((B, VOCAB), jnp.float32),
)


def kernel(x, word_emb, doc_emb, W_out, b_out):
    doc_ids = x[:, 0]
    word_ids = x[:, 1:].reshape(-1)
    pe, we = _gather_kernel()(doc_ids, word_ids, doc_emb, word_emb)
    concat = jnp.concatenate([pe, we.reshape(B, CTX * WDIM)], axis=1)
    b3 = jnp.pad(b_out, (0, VPAD - VOCAB)).reshape(NV, 1, VT)
    m, s = _stats_call(concat, W_out, b3)
    return _out_call(concat, W_out, b3, m, s)
